# baseline (speedup 1.0000x reference)
import jax
import jax.numpy as jnp
from jax import lax
from jax.experimental import pallas as pl
from jax.experimental.pallas import tpu as pltpu

_N = 512
_V = 100000
_W = 128

_NQ = 8
_G = _N // _NQ


def _tc_body(tgt_smem, probs_hbm, tgt2d_ref, dscr_ref, mle_ref,
             out_p, out_l, scratch, *sems):
    for q in range(_NQ):
        def issue(i, carry, q=q):
            r = i + q * _G
            t = tgt_smem[r]
            col = pl.multiple_of((t >> 7) * _W, _W)
            pltpu.make_async_copy(
                probs_hbm.at[pl.ds(r, 1), pl.ds(col, _W)],
                scratch.at[pl.ds(r, 1), :],
                sems[q],
            ).start()
            return carry

        lax.fori_loop(0, _G, issue, 0, unroll=8)

    rw = dscr_ref[...] + 0.2 * mle_ref[...]
    low = tgt2d_ref[...] & (_W - 1)
    acc = jnp.zeros((), jnp.float32)
    for q in range(_NQ):
        pltpu.make_async_copy(
            probs_hbm.at[pl.ds(0, _G), pl.ds(0, _W)],
            scratch.at[pl.ds(q * _G, _G), :],
            sems[q],
        ).wait()
        sl = pl.ds(q * _G, _G)
        lo, hi = q * _G, (q + 1) * _G
        val = jnp.take_along_axis(scratch[sl, :], low[lo:hi, :], axis=1)
        out_p[sl, :] = val
        acc = acc + jnp.sum(val * rw[lo:hi, :])
    out_l[0, 0] = -acc


@jax.jit
def _policy_update(p2d, targets, tgt2d, dscr2d, mle2d):
    return pl.pallas_call(
        _tc_body,
        in_specs=[
            pl.BlockSpec(memory_space=pltpu.SMEM),
            pl.BlockSpec(memory_space=pltpu.HBM),
            pl.BlockSpec(memory_space=pltpu.VMEM),
            pl.BlockSpec(memory_space=pltpu.VMEM),
            pl.BlockSpec(memory_space=pltpu.VMEM),
        ],
        out_specs=[
            pl.BlockSpec(memory_space=pltpu.VMEM),
            pl.BlockSpec(memory_space=pltpu.SMEM),
        ],
        out_shape=[
            jax.ShapeDtypeStruct((_N, 1), jnp.float32),
            jax.ShapeDtypeStruct((1, 1), jnp.float32),
        ],
        scratch_shapes=[
            pltpu.VMEM((_N, _W), jnp.float32),
        ] + [pltpu.SemaphoreType.DMA] * _NQ,
    )(targets, p2d, tgt2d, dscr2d, mle2d)


def kernel(probs, targets, dscr_rewards, mle_rewards):
    p2d = probs.reshape((_N, _V))
    out_p, out_l = _policy_update(
        p2d, targets, targets.reshape((_N, 1)),
        dscr_rewards.reshape((_N, 1)), mle_rewards.reshape((_N, 1)))
    return (out_p.reshape((_N,)), out_l[0, 0])


# speedup vs baseline: 1.0693x; 1.0693x over previous
import jax
import jax.numpy as jnp
from jax import lax
from jax.experimental import pallas as pl
from jax.experimental.pallas import tpu as pltpu

_N = 512
_V = 100000
_W = 128

_NQ = 4
_G = _N // _NQ


def _tc_body(tgt_smem, probs_hbm, tgt2d_ref, dscr_ref, mle_ref,
             out_p, out_l, scratch, *sems):
    for q in range(_NQ):
        def issue(i, carry, q=q):
            r = i + q * _G
            t = tgt_smem[r]
            col = pl.multiple_of((t >> 7) * _W, _W)
            pltpu.make_async_copy(
                probs_hbm.at[pl.ds(r, 1), pl.ds(col, _W)],
                scratch.at[pl.ds(r, 1), :],
                sems[q],
            ).start()
            return carry

        lax.fori_loop(0, _G, issue, 0, unroll=8)

    rw = dscr_ref[...] + 0.2 * mle_ref[...]
    low = tgt2d_ref[...] & (_W - 1)
    acc = jnp.zeros((), jnp.float32)
    for q in range(_NQ):
        pltpu.make_async_copy(
            probs_hbm.at[pl.ds(0, _G), pl.ds(0, _W)],
            scratch.at[pl.ds(q * _G, _G), :],
            sems[q],
        ).wait()
        sl = pl.ds(q * _G, _G)
        lo, hi = q * _G, (q + 1) * _G
        val = jnp.take_along_axis(scratch[sl, :], low[lo:hi, :], axis=1)
        out_p[sl, :] = val
        acc = acc + jnp.sum(val * rw[lo:hi, :])
    out_l[0, 0] = -acc


@jax.jit
def _policy_update(p2d, targets, tgt2d, dscr2d, mle2d):
    return pl.pallas_call(
        _tc_body,
        in_specs=[
            pl.BlockSpec(memory_space=pltpu.SMEM),
            pl.BlockSpec(memory_space=pltpu.HBM),
            pl.BlockSpec(memory_space=pltpu.VMEM),
            pl.BlockSpec(memory_space=pltpu.VMEM),
            pl.BlockSpec(memory_space=pltpu.VMEM),
        ],
        out_specs=[
            pl.BlockSpec(memory_space=pltpu.VMEM),
            pl.BlockSpec(memory_space=pltpu.SMEM),
        ],
        out_shape=[
            jax.ShapeDtypeStruct((_N, 1), jnp.float32),
            jax.ShapeDtypeStruct((1, 1), jnp.float32),
        ],
        scratch_shapes=[
            pltpu.VMEM((_N, _W), jnp.float32),
        ] + [pltpu.SemaphoreType.DMA] * _NQ,
    )(targets, p2d, tgt2d, dscr2d, mle2d)


def kernel(probs, targets, dscr_rewards, mle_rewards):
    p2d = probs.reshape((_N, _V))
    out_p, out_l = _policy_update(
        p2d, targets, targets.reshape((_N, 1)),
        dscr_rewards.reshape((_N, 1)), mle_rewards.reshape((_N, 1)))
    return (out_p.reshape((_N,)), out_l[0, 0])
